# Initial kernel scaffold; baseline (speedup 1.0000x reference)
import jax
import jax.numpy as jnp
from jax.experimental import pallas as pl


def kernel(user_idx, user_feats, item_idx, item_feats, user_table, item_table, uW1, ub1, uW2, ub2, uW3, ub3, iW1, ib1, iW2, ib2, iW3, ib3):
    raise NotImplementedError("write your pallas kernel here")


# speedup vs baseline: 2.4958x; 2.4958x over previous
import jax
import jax.numpy as jnp
from jax.experimental import pallas as pl
from jax.experimental.pallas import tpu as pltpu
from jax.experimental.pallas import tpu_sc as plsc

B = 16384
E = 128
FEAT = 64
H1, H2 = 512, 256
OUT = 128

GATHER_WINDOW = 128
BB = 1024


def _sc_gather(user_table, item_table, user_idx, item_idx):
    mesh = plsc.VectorSubcoreMesh(core_axis_name="c", subcore_axis_name="s")
    out_type = (
        jax.ShapeDtypeStruct((B, E), jnp.float32),
        jax.ShapeDtypeStruct((B, E), jnp.float32),
    )

    @pl.kernel(out_type=out_type, mesh=mesh)
    def gather_kernel(ut_hbm, it_hbm, ui_hbm, ii_hbm, uo_hbm, io_hbm):
        def body(ui_vmem, ii_vmem, uo_vmem, io_vmem):
            pltpu.sync_copy(ut_hbm.at[ui_vmem.at[0]], uo_vmem)
            pltpu.sync_copy(it_hbm.at[ii_vmem.at[0]], io_vmem)

        pltpu.emit_pipeline(
            body,
            grid=(B // GATHER_WINDOW,),
            in_specs=[
                pl.BlockSpec((1, GATHER_WINDOW), lambda i: (0, i)),
                pl.BlockSpec((1, GATHER_WINDOW), lambda i: (0, i)),
            ],
            out_specs=[
                pl.BlockSpec((GATHER_WINDOW, E), lambda i: (i, 0)),
                pl.BlockSpec((GATHER_WINDOW, E), lambda i: (i, 0)),
            ],
            core_axis_name=("c", "s"),
            dimension_semantics=(pltpu.PARALLEL,),
        )(ui_hbm, ii_hbm, uo_hbm, io_hbm)

    return gather_kernel(
        user_table,
        item_table,
        user_idx.reshape(1, B),
        item_idx.reshape(1, B),
    )


def _towers_body(ue, uf, ie, if_, uW1a, uW1b, ub1, uW2, ub2, uW3, ub3,
                 iW1a, iW1b, ib1, iW2, ib2, iW3, ib3, uo, io):
    def tower(emb, feat, W1a, W1b, b1, W2, b2, W3, b3, out_ref):
        xe = emb[...].astype(jnp.bfloat16)
        xf = feat[...].astype(jnp.bfloat16)
        h = jnp.dot(xe, W1a[...], preferred_element_type=jnp.float32)
        h = h + jnp.dot(xf, W1b[...], preferred_element_type=jnp.float32)
        h = jnp.maximum(h + b1[...], 0.0).astype(jnp.bfloat16)
        h = jnp.dot(h, W2[...], preferred_element_type=jnp.float32) + b2[...]
        h = jnp.maximum(h, 0.0).astype(jnp.bfloat16)
        o = jnp.dot(h, W3[...], preferred_element_type=jnp.float32) + b3[...]
        ss = jnp.sum(o * o, axis=-1, keepdims=True)
        out_ref[...] = o * jax.lax.rsqrt(jnp.maximum(ss, 1e-24))

    tower(ue, uf, uW1a, uW1b, ub1, uW2, ub2, uW3, ub3, uo)
    tower(ie, if_, iW1a, iW1b, ib1, iW2, ib2, iW3, ib3, io)


def _tc_towers(u_emb, user_feats, i_emb, item_feats, weights):
    nb = B // BB
    bspec = lambda shape: pl.BlockSpec(shape, lambda i: (i, 0))
    wspec = lambda shape: pl.BlockSpec(shape, lambda i: (0, 0))
    in_specs = [
        bspec((BB, E)), bspec((BB, FEAT)), bspec((BB, E)), bspec((BB, FEAT)),
        wspec((E, H1)), wspec((FEAT, H1)), wspec((1, H1)),
        wspec((H1, H2)), wspec((1, H2)),
        wspec((H2, OUT)), wspec((1, OUT)),
        wspec((E, H1)), wspec((FEAT, H1)), wspec((1, H1)),
        wspec((H1, H2)), wspec((1, H2)),
        wspec((H2, OUT)), wspec((1, OUT)),
    ]
    out_specs = (bspec((BB, OUT)), bspec((BB, OUT)))
    out_shape = (
        jax.ShapeDtypeStruct((B, OUT), jnp.float32),
        jax.ShapeDtypeStruct((B, OUT), jnp.float32),
    )
    return pl.pallas_call(
        _towers_body,
        grid=(nb,),
        in_specs=in_specs,
        out_specs=out_specs,
        out_shape=out_shape,
    )(u_emb, user_feats, i_emb, item_feats, *weights)


def kernel(user_idx, user_feats, item_idx, item_feats, user_table, item_table,
           uW1, ub1, uW2, ub2, uW3, ub3, iW1, ib1, iW2, ib2, iW3, ib3):
    u_emb, i_emb = _sc_gather(
        user_table, item_table,
        user_idx.astype(jnp.int32), item_idx.astype(jnp.int32))
    bf = jnp.bfloat16
    weights = (
        uW1[:E].astype(bf), uW1[E:].astype(bf), ub1.reshape(1, H1),
        uW2.astype(bf), ub2.reshape(1, H2),
        uW3.astype(bf), ub3.reshape(1, OUT),
        iW1[:E].astype(bf), iW1[E:].astype(bf), ib1.reshape(1, H1),
        iW2.astype(bf), ib2.reshape(1, H2),
        iW3.astype(bf), ib3.reshape(1, OUT),
    )
    return _tc_towers(u_emb, user_feats, i_emb, item_feats, weights)
